# baseline (speedup 1.0000x reference)
import functools
import math

import jax
import jax.numpy as jnp
from jax.experimental import pallas as pl
from jax.experimental.pallas import tpu as pltpu
from jax.experimental.pallas import tpu_sc as plsc

B = 8
S = 50000
G = 10000
J = 8
K = 32
D = 64
BN_INV = 1.0 / math.sqrt(1.0 + 1e-5)
SQRT_HALF = 1.0 / math.sqrt(2.0)

GP = 10240
GA = 256
NA = GP // GA
G_BLK = 512
N_BLK = GP // G_BLK

NC = 2
NS = 16
NW = NC * NS

MP = GP * J
M_TILE = MP // NW
M_CH = 640
NM_CH = M_TILE // M_CH

E = 160000
EP = 163840
E_TILE = EP // NS
E_CH = 640
N_CH = E_TILE // E_CH
G_TILE = GP // NS
B_PER_SC = B // NC


def _snp_gather(tx_hbm, tf_hbm, ids_hbm, gx_hbm, gf_hbm,
                gidx_v, rx_v, rf_v, sem):
    c = jax.lax.axis_index("c")
    s = jax.lax.axis_index("s")
    wid = s * NC + c
    for i in range(NM_CH):
        off = pl.multiple_of(wid * M_TILE + i * M_CH, 8)
        pltpu.sync_copy(ids_hbm.at[pl.ds(off, M_CH)], gidx_v)
        cpx = pltpu.async_copy(tx_hbm.at[gidx_v], rx_v, sem)
        cpf = pltpu.async_copy(tf_hbm.at[gidx_v], rf_v, sem)
        cpx.wait()
        cpf.wait()
        pltpu.sync_copy(rx_v, gx_hbm.at[pl.ds(off, M_CH)])
        pltpu.sync_copy(rf_v, gf_hbm.at[pl.ds(off, M_CH)])


def _stage_a(xg_ref, fg_ref, p_ref, emb_ref, h_ref, red_ref):
    red = xg_ref[0][:, None, :] * fg_ref[0][None, :, :]
    for j in range(1, J):
        red += xg_ref[j][:, None, :] * fg_ref[j][None, :, :]
    red_ref[...] = red
    h_ref[...] = jnp.broadcast_to(emb_ref[...][None, :, :], (B, GA, D))
    for k in range(K):
        h_ref[...] += red_ref[:, k, :][:, :, None] * p_ref[k][None, :, :]


def _gin_scatter(hflat_hbm, srcoff_hbm, dst_hbm, zeros_hbm, agg_hbm,
                 sidx0, sidx1, didx0, didx1, rows0, rows1, acc_sh,
                 sem0, sem1, semw0, semw1):
    c = jax.lax.axis_index("c")
    s = jax.lax.axis_index("s")
    gbase = pl.multiple_of(s * G_TILE, 8)
    sidx = (sidx0, sidx1)
    didx = (didx0, didx1)
    rows = (rows0, rows1)
    sems = (sem0, sem1)
    semw = (semw0, semw1)

    def load_chunk(b, i):
        j = i % 2
        off = pl.multiple_of(s * E_TILE + i * E_CH, 8)
        soff = pl.multiple_of(b * EP + s * E_TILE + i * E_CH, 8)
        pltpu.sync_copy(srcoff_hbm.at[pl.ds(soff, E_CH)], sidx[j])
        pltpu.sync_copy(dst_hbm.at[pl.ds(off, E_CH)], didx[j])
        return pltpu.async_copy(hflat_hbm.at[sidx[j]], rows[j], sems[j])

    for bi in range(B_PER_SC):
        b = c * B_PER_SC + bi
        pltpu.sync_copy(zeros_hbm, acc_sh.at[pl.ds(gbase, G_TILE)])
        plsc.subcore_barrier()
        cp = load_chunk(b, 0)
        adds = {}
        for i in range(N_CH):
            j = i % 2
            cp.wait()
            if i >= 1:
                adds[i - 1].wait()
            if i + 1 < N_CH:
                cp = load_chunk(b, i + 1)
            adds[i] = pltpu.async_copy(
                rows[j], acc_sh.at[didx[j]], semw[j], add=True)
        adds[N_CH - 1].wait()
        plsc.subcore_barrier()
        pltpu.sync_copy(acc_sh.at[pl.ds(gbase, G_TILE)],
                        agg_hbm.at[b].at[pl.ds(gbase, G_TILE)])
        plsc.subcore_barrier()


def _stage_b(h_ref, agg_ref, gin_ref, keyw_ref, keyb_ref, attw_ref,
             valw_ref, valb_ref, w_ref, gh_ref):
    @pl.when(pl.program_id(0) == 0)
    def _():
        gh_ref[...] = jnp.zeros((B, D), dtype=jnp.float32)

    gin_w = gin_ref[...]
    key_w = keyw_ref[...]
    key_b = keyb_ref[...]
    att_w = attw_ref[...]
    val_w = valw_ref[...]
    val_b = valb_ref[...]
    contribs = []
    for b in range(B):
        z = h_ref[b] + agg_ref[b]
        z = jnp.dot(z, gin_w, preferred_element_type=jnp.float32) * BN_INV
        z = 0.5 * z * (1.0 + jax.lax.erf(z * SQRT_HALF))
        keys = jnp.dot(z, key_w, preferred_element_type=jnp.float32) + key_b
        logits = jnp.sum(keys * att_w, axis=1, keepdims=True)
        row = (pl.program_id(0) * G_BLK
               + jax.lax.broadcasted_iota(jnp.int32, (G_BLK, 1), 0))
        w = jnp.where(row < G, jax.nn.sigmoid(logits), 0.0)
        v = jnp.dot(z, val_w, preferred_element_type=jnp.float32) + val_b
        w_ref[b, :] = w[:, 0]
        contribs.append(jnp.sum(w * v, axis=0))
    gh_ref[...] += jnp.stack(contribs)


def kernel(x, edge_index, snp_ids, gene2snp_len, pvalues, filters,
           gene_embedding, gene_proj, gin_w, key_w, key_b, att_w,
           val_w, val_b, pred_w, pred_b):
    ids_p = jnp.concatenate(
        [snp_ids, jnp.zeros(MP - G * J, dtype=snp_ids.dtype)])
    tx = x.T
    tf = filters.T

    gx, gf = functools.partial(
        pl.kernel,
        mesh=plsc.VectorSubcoreMesh(core_axis_name="c", subcore_axis_name="s"),
        out_type=[
            jax.ShapeDtypeStruct((MP, B), jnp.float32),
            jax.ShapeDtypeStruct((MP, K), jnp.float32),
        ],
        compiler_params=pltpu.CompilerParams(use_tc_tiling_on_sc=False),
        scratch_types=[
            pltpu.VMEM((M_CH,), jnp.int32),
            pltpu.VMEM((M_CH, B), jnp.float32),
            pltpu.VMEM((M_CH, K), jnp.float32),
            pltpu.SemaphoreType.DMA,
        ],
    )(_snp_gather)(tx, tf, ids_p)

    xg3 = gx.reshape(GP, J, B).transpose(1, 2, 0)
    fg3 = gf.reshape(GP, J, K).transpose(1, 2, 0)
    p = jnp.pad(gene_proj, ((0, GP - G), (0, 0))).reshape(
        GP, K, D).transpose(1, 0, 2)
    emb_p = jnp.pad(gene_embedding, ((0, GP - G), (0, 0)))

    h = pl.pallas_call(
        _stage_a,
        grid=(NA,),
        in_specs=[
            pl.BlockSpec((J, B, GA), lambda i: (0, 0, i)),
            pl.BlockSpec((J, K, GA), lambda i: (0, 0, i)),
            pl.BlockSpec((K, GA, D), lambda i: (0, i, 0)),
            pl.BlockSpec((GA, D), lambda i: (i, 0)),
        ],
        out_specs=pl.BlockSpec((B, GA, D), lambda i: (0, i, 0)),
        out_shape=jax.ShapeDtypeStruct((B, GP, D), jnp.float32),
        scratch_shapes=[pltpu.VMEM((B, K, GA), jnp.float32)],
    )(xg3, fg3, p, emb_p)

    src = edge_index[0]
    dst = edge_index[1]
    src_p = jnp.concatenate([src, jnp.zeros(EP - E, dtype=jnp.int32)])
    dst_p = jnp.concatenate(
        [dst, jnp.full(EP - E, G, dtype=jnp.int32)])
    src_off = (src_p[None, :] +
               (jnp.arange(B, dtype=jnp.int32) * GP)[:, None]).reshape(-1)
    hflat = h.reshape(B * GP, D)
    zeros_tile = jnp.zeros((G_TILE, D), jnp.float32)

    agg = functools.partial(
        pl.kernel,
        mesh=plsc.VectorSubcoreMesh(core_axis_name="c", subcore_axis_name="s"),
        out_type=jax.ShapeDtypeStruct((B, GP, D), jnp.float32),
        compiler_params=pltpu.CompilerParams(use_tc_tiling_on_sc=False),
        scratch_types=[
            pltpu.VMEM((E_CH,), jnp.int32),
            pltpu.VMEM((E_CH,), jnp.int32),
            pltpu.VMEM((E_CH,), jnp.int32),
            pltpu.VMEM((E_CH,), jnp.int32),
            pltpu.VMEM((E_CH, D), jnp.float32),
            pltpu.VMEM((E_CH, D), jnp.float32),
            pltpu.VMEM_SHARED((GP, D), jnp.float32),
            pltpu.SemaphoreType.DMA,
            pltpu.SemaphoreType.DMA,
            pltpu.SemaphoreType.DMA,
            pltpu.SemaphoreType.DMA,
        ],
    )(_gin_scatter)(hflat, src_off, dst_p, zeros_tile)

    w2, gh = pl.pallas_call(
        _stage_b,
        grid=(N_BLK,),
        in_specs=[
            pl.BlockSpec((B, G_BLK, D), lambda i: (0, i, 0)),
            pl.BlockSpec((B, G_BLK, D), lambda i: (0, i, 0)),
            pl.BlockSpec((D, D), lambda i: (0, 0)),
            pl.BlockSpec((D, D), lambda i: (0, 0)),
            pl.BlockSpec((1, D), lambda i: (0, 0)),
            pl.BlockSpec((1, D), lambda i: (0, 0)),
            pl.BlockSpec((D, D), lambda i: (0, 0)),
            pl.BlockSpec((1, D), lambda i: (0, 0)),
        ],
        out_specs=[
            pl.BlockSpec((B, G_BLK), lambda i: (0, i)),
            pl.BlockSpec((B, D), lambda i: (0, 0)),
        ],
        out_shape=[
            jax.ShapeDtypeStruct((B, GP), jnp.float32),
            jax.ShapeDtypeStruct((B, D), jnp.float32),
        ],
    )(h, agg, gin_w, key_w, key_b.reshape(1, D), att_w.reshape(1, D),
      val_w, val_b.reshape(1, D))

    pred = gh @ pred_w + pred_b
    return pred, w2[:, :G].reshape(-1, 1)


# speedup vs baseline: 1.0374x; 1.0374x over previous
import functools
import math

import jax
import jax.numpy as jnp
from jax.experimental import pallas as pl
from jax.experimental.pallas import tpu as pltpu
from jax.experimental.pallas import tpu_sc as plsc

B = 8
S = 50000
G = 10000
J = 8
K = 32
D = 64
BN_INV = 1.0 / math.sqrt(1.0 + 1e-5)
SQRT_HALF = 1.0 / math.sqrt(2.0)

GP = 10240
GA = 256
NA = GP // GA
G_BLK = 512
N_BLK = GP // G_BLK

NC = 2
NS = 16
NW = NC * NS

MP = GP * J
M_TILE = MP // NW
M_CH = 640
NM_CH = M_TILE // M_CH

E = 160000
EP = 163840
E_TILE = EP // NS
E_CH = 640
N_CH = E_TILE // E_CH
G_TILE = GP // NS
B_PER_SC = B // NC


def _snp_gather(tx_hbm, tf_hbm, ids_hbm, gx_hbm, gf_hbm,
                gidx_v, rx_v, rf_v, sem):
    c = jax.lax.axis_index("c")
    s = jax.lax.axis_index("s")
    wid = s * NC + c
    for i in range(NM_CH):
        off = pl.multiple_of(wid * M_TILE + i * M_CH, 8)
        pltpu.sync_copy(ids_hbm.at[pl.ds(off, M_CH)], gidx_v)
        cpx = pltpu.async_copy(tx_hbm.at[gidx_v], rx_v, sem)
        cpf = pltpu.async_copy(tf_hbm.at[gidx_v], rf_v, sem)
        cpx.wait()
        cpf.wait()
        pltpu.sync_copy(rx_v, gx_hbm.at[pl.ds(off, M_CH)])
        pltpu.sync_copy(rf_v, gf_hbm.at[pl.ds(off, M_CH)])


def _stage_a(xg_ref, fg_ref, p_ref, emb_ref, h_ref, red_ref):
    red = xg_ref[0][:, None, :] * fg_ref[0][None, :, :]
    for j in range(1, J):
        red += xg_ref[j][:, None, :] * fg_ref[j][None, :, :]
    red_ref[...] = red
    h_ref[...] = jnp.broadcast_to(emb_ref[...][None, :, :], (B, GA, D))
    for k in range(K):
        h_ref[...] += red_ref[:, k, :][:, :, None] * p_ref[k][None, :, :]


def _gin_scatter(hflat_hbm, srcoff_hbm, dst_hbm, zeros_hbm, agg_hbm,
                 sidx0, sidx1, didx0, didx1, rows0, rows1, acc_sh,
                 sem0, sem1):
    c = jax.lax.axis_index("c")
    s = jax.lax.axis_index("s")
    gbase = pl.multiple_of(s * G_TILE, 8)
    sidx = (sidx0, sidx1)
    didx = (didx0, didx1)
    rows = (rows0, rows1)
    sems = (sem0, sem1)

    def load_chunk(b, i):
        j = i % 2
        off = pl.multiple_of(s * E_TILE + i * E_CH, 8)
        soff = pl.multiple_of(b * EP + s * E_TILE + i * E_CH, 8)
        pltpu.sync_copy(srcoff_hbm.at[pl.ds(soff, E_CH)], sidx[j])
        pltpu.sync_copy(dst_hbm.at[pl.ds(off, E_CH)], didx[j])
        return pltpu.async_copy(hflat_hbm.at[sidx[j]], rows[j], sems[j])

    for bi in range(B_PER_SC):
        b = c * B_PER_SC + bi
        pltpu.sync_copy(zeros_hbm, acc_sh.at[pl.ds(gbase, G_TILE)])
        plsc.subcore_barrier()
        cp = load_chunk(b, 0)
        for i in range(N_CH):
            j = i % 2
            nxt = load_chunk(b, i + 1) if i + 1 < N_CH else None
            cp.wait()
            pltpu.sync_copy(rows[j], acc_sh.at[didx[j]], add=True)
            cp = nxt
        plsc.subcore_barrier()
        pltpu.sync_copy(acc_sh.at[pl.ds(gbase, G_TILE)],
                        agg_hbm.at[b].at[pl.ds(gbase, G_TILE)])
        plsc.subcore_barrier()


def _stage_b(h_ref, agg_ref, gin_ref, keyw_ref, keyb_ref, attw_ref,
             valw_ref, valb_ref, w_ref, gh_ref):
    @pl.when(pl.program_id(0) == 0)
    def _():
        gh_ref[...] = jnp.zeros((B, D), dtype=jnp.float32)

    gin_w = gin_ref[...]
    key_w = keyw_ref[...]
    key_b = keyb_ref[...]
    att_w = attw_ref[...]
    val_w = valw_ref[...]
    val_b = valb_ref[...]
    contribs = []
    for b in range(B):
        z = h_ref[b] + agg_ref[b]
        z = jnp.dot(z, gin_w, preferred_element_type=jnp.float32) * BN_INV
        z = 0.5 * z * (1.0 + jax.lax.erf(z * SQRT_HALF))
        keys = jnp.dot(z, key_w, preferred_element_type=jnp.float32) + key_b
        logits = jnp.sum(keys * att_w, axis=1, keepdims=True)
        row = (pl.program_id(0) * G_BLK
               + jax.lax.broadcasted_iota(jnp.int32, (G_BLK, 1), 0))
        w = jnp.where(row < G, jax.nn.sigmoid(logits), 0.0)
        v = jnp.dot(z, val_w, preferred_element_type=jnp.float32) + val_b
        w_ref[b, :] = w[:, 0]
        contribs.append(jnp.sum(w * v, axis=0))
    gh_ref[...] += jnp.stack(contribs)


def kernel(x, edge_index, snp_ids, gene2snp_len, pvalues, filters,
           gene_embedding, gene_proj, gin_w, key_w, key_b, att_w,
           val_w, val_b, pred_w, pred_b):
    ids_p = jnp.concatenate(
        [snp_ids, jnp.zeros(MP - G * J, dtype=snp_ids.dtype)])
    tx = x.T
    tf = filters.T

    gx, gf = functools.partial(
        pl.kernel,
        mesh=plsc.VectorSubcoreMesh(core_axis_name="c", subcore_axis_name="s"),
        out_type=[
            jax.ShapeDtypeStruct((MP, B), jnp.float32),
            jax.ShapeDtypeStruct((MP, K), jnp.float32),
        ],
        compiler_params=pltpu.CompilerParams(use_tc_tiling_on_sc=False),
        scratch_types=[
            pltpu.VMEM((M_CH,), jnp.int32),
            pltpu.VMEM((M_CH, B), jnp.float32),
            pltpu.VMEM((M_CH, K), jnp.float32),
            pltpu.SemaphoreType.DMA,
        ],
    )(_snp_gather)(tx, tf, ids_p)

    xg3 = gx.reshape(GP, J, B).transpose(1, 2, 0)
    fg3 = gf.reshape(GP, J, K).transpose(1, 2, 0)
    p = jnp.pad(gene_proj, ((0, GP - G), (0, 0))).reshape(
        GP, K, D).transpose(1, 0, 2)
    emb_p = jnp.pad(gene_embedding, ((0, GP - G), (0, 0)))

    h = pl.pallas_call(
        _stage_a,
        grid=(NA,),
        in_specs=[
            pl.BlockSpec((J, B, GA), lambda i: (0, 0, i)),
            pl.BlockSpec((J, K, GA), lambda i: (0, 0, i)),
            pl.BlockSpec((K, GA, D), lambda i: (0, i, 0)),
            pl.BlockSpec((GA, D), lambda i: (i, 0)),
        ],
        out_specs=pl.BlockSpec((B, GA, D), lambda i: (0, i, 0)),
        out_shape=jax.ShapeDtypeStruct((B, GP, D), jnp.float32),
        scratch_shapes=[pltpu.VMEM((B, K, GA), jnp.float32)],
    )(xg3, fg3, p, emb_p)

    src = edge_index[0]
    dst = edge_index[1]
    src_p = jnp.concatenate([src, jnp.zeros(EP - E, dtype=jnp.int32)])
    dst_p = jnp.concatenate(
        [dst, jnp.full(EP - E, G, dtype=jnp.int32)])
    src_off = (src_p[None, :] +
               (jnp.arange(B, dtype=jnp.int32) * GP)[:, None]).reshape(-1)
    hflat = h.reshape(B * GP, D)
    zeros_tile = jnp.zeros((G_TILE, D), jnp.float32)

    agg = functools.partial(
        pl.kernel,
        mesh=plsc.VectorSubcoreMesh(core_axis_name="c", subcore_axis_name="s"),
        out_type=jax.ShapeDtypeStruct((B, GP, D), jnp.float32),
        compiler_params=pltpu.CompilerParams(use_tc_tiling_on_sc=False),
        scratch_types=[
            pltpu.VMEM((E_CH,), jnp.int32),
            pltpu.VMEM((E_CH,), jnp.int32),
            pltpu.VMEM((E_CH,), jnp.int32),
            pltpu.VMEM((E_CH,), jnp.int32),
            pltpu.VMEM((E_CH, D), jnp.float32),
            pltpu.VMEM((E_CH, D), jnp.float32),
            pltpu.VMEM_SHARED((GP, D), jnp.float32),
            pltpu.SemaphoreType.DMA,
            pltpu.SemaphoreType.DMA,
        ],
    )(_gin_scatter)(hflat, src_off, dst_p, zeros_tile)

    w2, gh = pl.pallas_call(
        _stage_b,
        grid=(N_BLK,),
        in_specs=[
            pl.BlockSpec((B, G_BLK, D), lambda i: (0, i, 0)),
            pl.BlockSpec((B, G_BLK, D), lambda i: (0, i, 0)),
            pl.BlockSpec((D, D), lambda i: (0, 0)),
            pl.BlockSpec((D, D), lambda i: (0, 0)),
            pl.BlockSpec((1, D), lambda i: (0, 0)),
            pl.BlockSpec((1, D), lambda i: (0, 0)),
            pl.BlockSpec((D, D), lambda i: (0, 0)),
            pl.BlockSpec((1, D), lambda i: (0, 0)),
        ],
        out_specs=[
            pl.BlockSpec((B, G_BLK), lambda i: (0, i)),
            pl.BlockSpec((B, D), lambda i: (0, 0)),
        ],
        out_shape=[
            jax.ShapeDtypeStruct((B, GP), jnp.float32),
            jax.ShapeDtypeStruct((B, D), jnp.float32),
        ],
    )(h, agg, gin_w, key_w, key_b.reshape(1, D), att_w.reshape(1, D),
      val_w, val_b.reshape(1, D))

    pred = gh @ pred_w + pred_b
    return pred, w2[:, :G].reshape(-1, 1)
